# baseline (speedup 1.0000x reference)
import functools

import jax
import jax.numpy as jnp
from jax import lax
from jax.experimental import pallas as pl
from jax.experimental.pallas import tpu as pltpu
from jax.experimental.pallas import tpu_sc as plsc

MIN_POS = -1.2
MAX_POS = 0.6
MAX_SPEED = 0.07

N = 16384
NC = 2
NS = 16
NW = NC * NS
CHUNK = N // NW
LANES = 16
GROUPS = CHUNK // LANES
IDX_BLK = 128
NBLK = CHUNK // IDX_BLK

_mesh = plsc.VectorSubcoreMesh(core_axis_name="c", subcore_axis_name="s")


@functools.partial(
    pl.kernel,
    mesh=_mesh,
    out_type=jax.ShapeDtypeStruct((N,), jnp.float32),
    scratch_types=[
        pltpu.VMEM((2 * CHUNK,), jnp.float32),
        pltpu.VMEM((CHUNK,), jnp.int32),
        pltpu.VMEM((CHUNK,), jnp.float32),
        pltpu.SemaphoreType.DMA,
    ],
    compiler_params=pltpu.CompilerParams(needs_layout_passes=False),
)
def _sc_lookup(inp_hbm, data_hbm, out_hbm, inp_v, idx_v, out_v, sem):
    wid = lax.axis_index("s") * NC + lax.axis_index("c")
    base = wid * CHUNK

    pltpu.sync_copy(inp_hbm.at[pl.ds(base * 2, 2 * CHUNK)], inp_v)

    b0 = jnp.float32(-MIN_POS)
    b1 = jnp.float32(MAX_SPEED)
    m0 = jnp.float32(1023.999 / (MAX_POS - MIN_POS))
    m1 = jnp.float32(1023.999 / (2.0 * MAX_SPEED))

    lanes = lax.iota(jnp.int32, LANES)
    even = 2 * lanes
    odd = even + 1

    for g in range(GROUPS):
        off = g * 2 * LANES
        pos = plsc.load_gather(inp_v, [off + even])
        vel = plsc.load_gather(inp_v, [off + odd])
        r = ((pos + b0) * m0).astype(jnp.int32)
        c = ((vel + b1) * m1).astype(jnp.int32)
        idx_v[pl.ds(g * LANES, LANES)] = r * 1024 + c

    copies = [
        pltpu.async_copy(
            data_hbm.at[idx_v.at[pl.ds(j * IDX_BLK, IDX_BLK)]],
            out_v.at[pl.ds(j * IDX_BLK, IDX_BLK)],
            sem,
        )
        for j in range(NBLK)
    ]
    for cp in copies:
        cp.wait()

    pltpu.sync_copy(out_v, out_hbm.at[pl.ds(base, CHUNK)])


def kernel(inp, data):
    return _sc_lookup(inp.reshape(-1), data.reshape(-1))
